# Initial kernel scaffold; baseline (speedup 1.0000x reference)
import jax
import jax.numpy as jnp
from jax.experimental import pallas as pl


def kernel(visit_concept_orders, pos_encoding):
    raise NotImplementedError("write your pallas kernel here")


# speedup vs baseline: 5.1410x; 5.1410x over previous
import functools

import jax
import jax.numpy as jnp
from jax import lax
from jax.experimental import pallas as pl
from jax.experimental.pallas import tpu as pltpu
from jax.experimental.pallas import tpu_sc as plsc

B, L, D = 1024, 200, 128
LANE = 16
_info = plsc.get_sparse_core_info()
NC, NS = _info.num_cores, _info.num_subcores
NW = NC * NS
ROWS_PER_W = B // NW
CH0 = 112
CH1 = L - CH0

_mesh = plsc.VectorSubcoreMesh(core_axis_name="c", subcore_axis_name="s")

_GATHER_DNUMS = lax.GatherDimensionNumbers(
    offset_dims=(), collapsed_slice_dims=(0,), start_index_map=(0,))


def _lane_permute(x, perm):
    return lax.gather(
        x, perm[:, None], _GATHER_DNUMS, slice_sizes=(1,),
        mode=lax.GatherScatterMode.PROMISE_IN_BOUNDS)


@functools.partial(
    pl.kernel,
    out_type=jax.ShapeDtypeStruct((B, L, D), jnp.float32),
    mesh=_mesh,
    scratch_types=[
        pltpu.VMEM((ROWS_PER_W, L), jnp.int32),
        pltpu.VMEM((L,), jnp.int32),
        pltpu.VMEM((L, D), jnp.float32),
        pltpu.SemaphoreType.DMA,
    ],
)
def _pe_kernel(vco_hbm, table_hbm, out_hbm, idx_v, rel_v, rows_v, sem):
    wid = lax.axis_index("s") * NC + lax.axis_index("c")
    base = wid * ROWS_PER_W
    pltpu.sync_copy(vco_hbm.at[pl.ds(base, ROWS_PER_W)], idx_v)

    def row_body(r, carry):
        m = idx_v[r, pl.ds(0, LANE)]
        for k in range(1, L // LANE):
            m = jnp.minimum(m, idx_v[r, pl.ds(k * LANE, LANE)])
        m = jnp.minimum(m, idx_v[r, pl.ds(L - LANE, LANE)])
        lanes = lax.iota(jnp.int32, LANE)
        for sh in (8, 4, 2, 1):
            perm = lax.rem(lanes + sh, LANE)
            m = jnp.minimum(m, _lane_permute(m, perm))
        mn = m
        for k in range(L // LANE):
            rel_v[pl.ds(k * LANE, LANE)] = jnp.abs(
                idx_v[r, pl.ds(k * LANE, LANE)] - mn)
        rel_v[pl.ds(L - LANE, LANE)] = jnp.abs(
            idx_v[r, pl.ds(L - LANE, LANE)] - mn)
        cp0 = pltpu.async_copy(
            table_hbm.at[rel_v.at[pl.ds(0, CH0)]],
            rows_v.at[pl.ds(0, CH0)], sem)
        cp1 = pltpu.async_copy(
            table_hbm.at[rel_v.at[pl.ds(CH0, CH1)]],
            rows_v.at[pl.ds(CH0, CH1)], sem)
        cp0.wait()
        cp1.wait()
        pltpu.sync_copy(rows_v, out_hbm.at[base + r])
        return carry

    lax.fori_loop(0, ROWS_PER_W, row_body, 0)


def kernel(visit_concept_orders, pos_encoding):
    return _pe_kernel(visit_concept_orders.astype(jnp.int32), pos_encoding)
